# Initial kernel scaffold; baseline (speedup 1.0000x reference)
import jax
import jax.numpy as jnp
from jax.experimental import pallas as pl


def kernel(idx, weight):
    raise NotImplementedError("write your pallas kernel here")


# speedup vs baseline: 1.8711x; 1.8711x over previous
import functools

import jax
import jax.numpy as jnp
from jax import lax
from jax.experimental import pallas as pl
from jax.experimental.pallas import tpu as pltpu
from jax.experimental.pallas import tpu_sc as plsc

D = 64
L = 128
NW = 32
K = 4


def _gather_body(idx_hbm, tab_hbm, out_hbm, idx_v, rows_v, sem_i, sem_g, sem_o):
    rows_total = idx_hbm.shape[0]
    rows_per_w = rows_total // NW
    nblk = rows_per_w // K
    wid = lax.axis_index("s") * 2 + lax.axis_index("c")
    base = wid * rows_per_w

    pltpu.async_copy(idx_hbm.at[pl.ds(base, K)], idx_v.at[0], sem_i).wait()

    def step(b, s):

        @pl.when(b >= 2)
        def _():
            pltpu.make_async_copy(
                rows_v.at[s], out_hbm.at[pl.ds(base + (b - 2) * K, K)],
                sem_o).wait()

        @pl.when(b + 1 < nblk)
        def _():
            pltpu.async_copy(
                idx_hbm.at[pl.ds(base + (b + 1) * K, K)], idx_v.at[1 - s],
                sem_i)

        gathers = [
            pltpu.async_copy(
                tab_hbm.at[idx_v.at[s].at[j]], rows_v.at[s].at[j], sem_g)
            for j in range(K)
        ]
        for cp in gathers:
            cp.wait()

        pltpu.async_copy(
            rows_v.at[s], out_hbm.at[pl.ds(base + b * K, K)], sem_o)

        @pl.when(b + 1 < nblk)
        def _():
            pltpu.make_async_copy(
                idx_hbm.at[pl.ds(base + (b + 1) * K, K)], idx_v.at[1 - s],
                sem_i).wait()

    def pair(p, carry):
        step(p * 2, 0)
        step(p * 2 + 1, 1)
        return carry

    lax.fori_loop(0, nblk // 2, pair, 0, unroll=False)

    for tail in (nblk - 2, nblk - 1):
        pltpu.make_async_copy(
            rows_v.at[tail % 2], out_hbm.at[pl.ds(base + tail * K, K)],
            sem_o).wait()


def kernel(idx, weight):
    B, S = idx.shape
    total = B * S
    rows_total = total // L
    idx_flat = idx.reshape(rows_total, L).astype(jnp.int32)

    grid_kernel = functools.partial(
        pl.kernel,
        out_type=jax.ShapeDtypeStruct((rows_total, L, D), jnp.float32),
        mesh=plsc.VectorSubcoreMesh(core_axis_name="c", subcore_axis_name="s"),
        scratch_types=[
            pltpu.VMEM((2, K, L), jnp.int32),
            pltpu.VMEM((2, K, L, D), jnp.float32),
            pltpu.SemaphoreType.DMA,
            pltpu.SemaphoreType.DMA,
            pltpu.SemaphoreType.DMA,
        ],
        compiler_params=pltpu.CompilerParams(use_tc_tiling_on_sc=False),
    )
    out = grid_kernel(_gather_body)(idx_flat, weight)
    return out.reshape(B, S, D)
